# baseline (speedup 1.0000x reference)
import functools

import jax
import jax.numpy as jnp
from jax.experimental import pallas as pl


def _encoder_body(rect_ref, recog_ref, oq_ref,
                  wr6_ref, brect_ref, wrecog_ref, brecog_ref,
                  w1_ref, c1_ref, w2_ref, c2_ref,
                  out_ref, *, t_inv, sem_w, out_w):
    avg = jnp.dot(rect_ref[...], wr6_ref[...],
                  preferred_element_type=jnp.float32) + brect_ref[...]
    rmean = jnp.sum(recog_ref[...], axis=1) * t_inv
    rec = jnp.dot(rmean, wrecog_ref[...],
                  preferred_element_type=jnp.float32) + brecog_ref[...]
    x = jnp.concatenate([avg, rec], axis=1)
    h = jnp.maximum(jnp.dot(x, w1_ref[...],
                            preferred_element_type=jnp.float32) + c1_ref[...], 0.0)
    sem = jnp.maximum(jnp.dot(h, w2_ref[...],
                              preferred_element_type=jnp.float32) + c2_ref[...], 0.0)

    oq = oq_ref[...] * (1.0 / 1024.0)
    c = [oq[:, k:k + 1] for k in range(8)]
    d1x = (c[2] + c[4] - c[0] - c[6]) * 0.5
    d1y = (c[3] + c[5] - c[1] - c[7]) * 0.5
    wd = jnp.sqrt(d1x * d1x + d1y * d1y)
    den = jnp.maximum(wd, 1e-6)
    d1xn = d1x / den
    d1yn = d1y / den
    hx = (c[6] - c[0] + c[4] - c[2]) * 0.5
    hy = (c[7] - c[1] + c[5] - c[3]) * 0.5
    hts = jnp.sqrt(hx * hx + hy * hy)
    geom = jnp.concatenate([oq, d1xn, d1yn, -d1yn, d1xn, wd, hts], axis=1)

    pad = out_w - sem_w - 14
    out_ref[...] = jnp.concatenate(
        [sem[:, :sem_w], geom,
         jnp.zeros((geom.shape[0], pad), jnp.float32)], axis=1)


def kernel(rectified_quads, original_quads, region_counts, recog_features,
           W_rect, b_rect, W_recog, b_recog,
           W1, b1, g1, bt1, W2, b2, g2, bt2):
    del region_counts
    rectified_quads = rectified_quads.astype(jnp.float32)
    recog_features = recog_features.astype(jnp.float32)
    n, ch = rectified_quads.shape[0], rectified_quads.shape[1]
    sp = rectified_quads.shape[2] * rectified_quads.shape[3]
    t = recog_features.shape[1]
    sem_w = W2.shape[1]
    out_w = 128

    rect_flat = rectified_quads.reshape(n, ch * sp)
    oq8 = original_quads.reshape(n, 8)

    wr6 = jnp.repeat(W_rect * (1.0 / sp), sp, axis=0)
    inv = 1.0 / jnp.sqrt(1.0 + 1e-5)
    s1 = g1 * inv
    s2 = g2 * inv
    w1s = W1 * s1[None, :]
    c1 = (b1 * s1 + bt1).reshape(1, -1)
    w2s = W2 * s2[None, :]
    c2 = b2 * s2 + bt2
    w2p = jnp.zeros((W2.shape[0], out_w), jnp.float32).at[:, :sem_w].set(w2s)
    c2p = jnp.zeros((1, out_w), jnp.float32).at[0, :sem_w].set(c2)
    brect = b_rect.reshape(1, -1)
    brecog = b_recog.reshape(1, -1)

    nt = 400
    assert n % nt == 0, (n, nt)
    grid = n // nt

    body = functools.partial(_encoder_body, t_inv=1.0 / t,
                             sem_w=sem_w, out_w=out_w)
    rep = lambda i: (0, 0)
    out = pl.pallas_call(
        body,
        grid=(grid,),
        in_specs=[
            pl.BlockSpec((nt, ch * sp), lambda i: (i, 0)),
            pl.BlockSpec((nt, t, recog_features.shape[2]), lambda i: (i, 0, 0)),
            pl.BlockSpec((nt, 8), lambda i: (i, 0)),
            pl.BlockSpec(wr6.shape, rep),
            pl.BlockSpec(brect.shape, rep),
            pl.BlockSpec(W_recog.shape, rep),
            pl.BlockSpec(brecog.shape, rep),
            pl.BlockSpec(w1s.shape, rep),
            pl.BlockSpec(c1.shape, rep),
            pl.BlockSpec(w2p.shape, rep),
            pl.BlockSpec(c2p.shape, rep),
        ],
        out_specs=pl.BlockSpec((nt, out_w), lambda i: (i, 0)),
        out_shape=jax.ShapeDtypeStruct((n, out_w), jnp.float32),
    )(rect_flat, recog_features, oq8,
      wr6, brect, W_recog, brecog, w1s, c1, w2p, c2p)
    return out[:, :sem_w + 14]
